# baseline (speedup 1.0000x reference)
import functools

import jax
import jax.numpy as jnp
from jax import lax
from jax.experimental import pallas as pl
from jax.experimental.pallas import tpu as pltpu
from jax.experimental.pallas import tpu_sc as plsc

N = 10000
E = 320000
D = 128
H = 4
OC = 32
NET = 8
NNT = 8
NEG = 0.2

NP = 10240
NC = 2
NS = 16
RPT = NP // NS
NW = NC * NS
EPW = E // NW
CH = 80
NCHUNK = EPW // CH

F32 = jnp.float32
I32 = jnp.int32
HIGH = jax.lax.Precision.HIGHEST


def _dotT(a, b):
    return jax.lax.dot_general(a, b, (((1,), (1,)), ((), ())), precision=HIGH)


def _dot(a, b):
    return jax.lax.dot_general(a, b, (((1,), (0,)), ((), ())), precision=HIGH)


def _pre_body(x_ref, nt_ref, wq_ref, wk_ref, wv_ref, ati_ref, atj_ref,
              ntab_ref, etab_ref, aiaj_ref, xv_ref, ea_ref, ev_ref):
    x = x_ref[...]
    nt = nt_ref[...]
    onehot = (nt == lax.broadcasted_iota(I32, (N, NNT), 1)).astype(F32)
    xp = x + _dot(onehot, ntab_ref[...])

    row = lax.broadcasted_iota(I32, (H * OC, H), 0)
    col = lax.broadcasted_iota(I32, (H * OC, H), 1)
    blkmask = ((row // OC) == col).astype(F32)
    AiM = blkmask * ati_ref[...]
    AjM = blkmask * atj_ref[...]

    Qi = jax.lax.dot_general(wq_ref[...], AiM, (((0,), (0,)), ((), ())),
                             precision=HIGH)
    Kj = jax.lax.dot_general(wk_ref[...], AjM, (((0,), (0,)), ((), ())),
                             precision=HIGH)

    ai = _dot(xp, Qi)
    aj = _dot(xp, Kj)
    aiaj_ref[...] = jnp.concatenate(
        [ai, aj, jnp.zeros((N, 8), F32)], axis=1)
    xv_ref[...] = _dotT(xp, wv_ref[...])
    ea_ref[...] = _dot(etab_ref[...], Kj)
    ev_ref[...] = _dotT(etab_ref[...], wv_ref[...])


def _precompute(x, node_type, Wq, Wk, Wv, att_i, att_j, node_table, edge_table):
    return pl.pallas_call(
        _pre_body,
        out_shape=[
            jax.ShapeDtypeStruct((N, 16), F32),
            jax.ShapeDtypeStruct((N, D), F32),
            jax.ShapeDtypeStruct((NET, H), F32),
            jax.ShapeDtypeStruct((NET, D), F32),
        ],
    )(x, node_type.reshape(N, 1), Wq, Wk, Wv,
      att_i.reshape(H * OC, 1), att_j.reshape(H * OC, 1),
      node_table, edge_table)


def _sc_body(src_hbm, dst_hbm, et_hbm, aiaj_hbm, ea_hbm, xv_hbm,
             outp_hbm, cp_hbm,
             ea_v, src_v, dst_v, et_v, aa_d, aa_s, wrow_v, rows_v,
             zv_v, zc_v, out_acc, c_acc, sem):
    cid = lax.axis_index("c")
    sid = lax.axis_index("s")
    wid = sid * NC + cid
    wbase = wid * EPW

    pltpu.sync_copy(ea_hbm, ea_v)

    for r in range(32):
        for j in range(8):
            zv_v[r, pl.ds(j * 16, 16)] = jnp.zeros((16,), F32)
    for r in range(32):
        for j in range(2):
            zc_v[r, pl.ds(j * 16, 16)] = jnp.zeros((16,), F32)
    for r in range(CH):
        for j in range(2):
            wrow_v[r, pl.ds(j * 16, 16)] = jnp.zeros((16,), F32)
    for i in range(RPT // 32):
        pltpu.sync_copy(zv_v, out_acc.at[pl.ds(sid * RPT + i * 32, 32)])
        pltpu.sync_copy(zc_v, c_acc.at[pl.ds(sid * RPT + i * 32, 32)])
    plsc.subcore_barrier()

    lane = lax.iota(I32, 16)

    def chunk(t, carry):
        off = wbase + t * CH
        pltpu.sync_copy(src_hbm.at[pl.ds(off, CH)], src_v)
        pltpu.sync_copy(dst_hbm.at[pl.ds(off, CH)], dst_v)
        pltpu.sync_copy(et_hbm.at[pl.ds(off, CH)], et_v)
        d_rows = pltpu.async_copy(xv_hbm.at[src_v], rows_v, sem)
        d_ai = pltpu.async_copy(aiaj_hbm.at[dst_v], aa_d, sem)
        d_aj = pltpu.async_copy(aiaj_hbm.at[src_v], aa_s, sem)
        d_rows.wait()
        d_ai.wait()
        d_aj.wait()

        for j in range(CH // 16):
            sl = pl.ds(j * 16, 16)
            s16 = src_v[sl]
            d16 = dst_v[sl]
            e16 = et_v[sl]
            kvec = lane + j * 16
            live = s16 != d16
            for h in range(H):
                av = plsc.load_gather(aa_d, [kvec, jnp.full((16,), h, I32)])
                bv = plsc.load_gather(aa_s, [kvec, jnp.full((16,), h + H, I32)])
                gv = plsc.load_gather(ea_v, [e16 * H + h])
                a = av + bv + gv
                a = jnp.where(a >= 0, a, NEG * a)
                w = jnp.where(live, jnp.exp(a), jnp.zeros((16,), F32))
                plsc.store_scatter(wrow_v, [kvec, e16 * H + h], w)
                for c in range(h * OC, (h + 1) * OC):
                    csplat = jnp.full((16,), c, I32)
                    rv = plsc.load_gather(rows_v, [kvec, csplat])
                    plsc.store_scatter(rows_v, [kvec, csplat], rv * w)

        pltpu.sync_copy(rows_v, out_acc.at[dst_v], add=True)
        pltpu.sync_copy(wrow_v, c_acc.at[dst_v], add=True)

        for j in range(CH // 16):
            sl = pl.ds(j * 16, 16)
            e16 = et_v[sl]
            kvec = lane + j * 16
            z16 = jnp.zeros((16,), F32)
            for h in range(H):
                plsc.store_scatter(wrow_v, [kvec, e16 * H + h], z16)
        return carry

    lax.fori_loop(0, NCHUNK, chunk, 0)
    plsc.subcore_barrier()

    pltpu.sync_copy(out_acc.at[pl.ds(sid * RPT, RPT)],
                    outp_hbm.at[cid, pl.ds(sid * RPT, RPT)])
    pltpu.sync_copy(c_acc.at[pl.ds(sid * RPT, RPT)],
                    cp_hbm.at[cid, pl.ds(sid * RPT, RPT)])


def _edge_phase(src, dst, et, aiaj, ea_flat, xv):
    mesh = plsc.VectorSubcoreMesh(core_axis_name="c", subcore_axis_name="s",
                                  num_cores=NC, num_subcores=NS)
    fn = pl.kernel(
        _sc_body,
        out_type=[
            jax.ShapeDtypeStruct((NC, NP, D), F32),
            jax.ShapeDtypeStruct((NC, NP, NET * H), F32),
        ],
        mesh=mesh,
        compiler_params=pltpu.CompilerParams(use_tc_tiling_on_sc=False,
                                             needs_layout_passes=False),
        scratch_types=[
            pltpu.VMEM((NET * H,), F32),
            pltpu.VMEM((CH,), I32),
            pltpu.VMEM((CH,), I32),
            pltpu.VMEM((CH,), I32),
            pltpu.VMEM((CH, 16), F32),
            pltpu.VMEM((CH, 16), F32),
            pltpu.VMEM((CH, NET * H), F32),
            pltpu.VMEM((CH, D), F32),
            pltpu.VMEM((32, D), F32),
            pltpu.VMEM((32, NET * H), F32),
            pltpu.VMEM_SHARED((NP, D), F32),
            pltpu.VMEM_SHARED((NP, NET * H), F32),
            pltpu.SemaphoreType.DMA,
        ],
    )
    return fn(src, dst, et, aiaj, ea_flat, xv)


MB = 2000


def _merge_body(p_ref, c_ref, aiaj_ref, xv_ref, ea_ref, ev_ref,
                bias_ref, out_ref):
    csum = c_ref[0] + c_ref[1]
    ii = lax.broadcasted_iota(I32, (NET * H, D), 0)
    jj = lax.broadcasted_iota(I32, (NET * H, D), 1)
    HM = ((ii % H) == (jj // OC)).astype(F32)
    ri = lax.broadcasted_iota(I32, (NET * H, NET), 0)
    rj = lax.broadcasted_iota(I32, (NET * H, NET), 1)
    REP = ((ri // H) == rj).astype(F32)
    rep_ev = _dot(REP, ev_ref[...])
    M = HM * rep_ev

    out2 = _dot(csum, M)
    sfull = _dot(csum, HM)

    e4i = lax.broadcasted_iota(I32, (H, D), 0)
    e4j = lax.broadcasted_iota(I32, (H, D), 1)
    E4 = (e4i == (e4j // OC)).astype(F32)
    aiF = _dot(aiaj_ref[:, 0:H], E4)
    ajF = _dot(aiaj_ref[:, H:2 * H], E4)
    ea0F = _dot(ea_ref[0:1, :], E4)

    aself = aiF + ajF + ea0F
    aself = jnp.where(aself >= 0, aself, NEG * aself)
    wself = jnp.exp(aself)

    xv = xv_ref[...]
    ev0 = ev_ref[0:1, :]
    numer = p_ref[0] + p_ref[1] + out2 + wself * (xv + ev0)
    denom = sfull + wself + 1e-16
    out_ref[...] = numer / denom + bias_ref[...]


def _merge(p, c, aiaj, xv, ea, ev, bias2d):
    return pl.pallas_call(
        _merge_body,
        grid=(N // MB,),
        in_specs=[
            pl.BlockSpec((NC, MB, D), lambda i: (0, i, 0)),
            pl.BlockSpec((NC, MB, NET * H), lambda i: (0, i, 0)),
            pl.BlockSpec((MB, 16), lambda i: (i, 0)),
            pl.BlockSpec((MB, D), lambda i: (i, 0)),
            pl.BlockSpec((NET, H), lambda i: (0, 0)),
            pl.BlockSpec((NET, D), lambda i: (0, 0)),
            pl.BlockSpec((1, D), lambda i: (0, 0)),
        ],
        out_specs=pl.BlockSpec((MB, D), lambda i: (i, 0)),
        out_shape=jax.ShapeDtypeStruct((N, D), F32),
    )(p[:, :N, :], c[:, :N, :], aiaj, xv, ea, ev, bias2d)


def kernel(x, edge_index, node_type, edge_type, Wq, Wk, Wv, att_i, att_j,
           bias, node_table, edge_table):
    aiaj, xv, ea, ev = _precompute(
        x, node_type, Wq, Wk, Wv, att_i, att_j, node_table, edge_table)
    p, c = _edge_phase(
        edge_index[0], edge_index[1], edge_type,
        aiaj, ea.reshape(-1), xv)
    return _merge(p, c, aiaj, xv, ea, ev, bias.reshape(1, D))


# speedup vs baseline: 2.1820x; 2.1820x over previous
import functools

import jax
import jax.numpy as jnp
from jax import lax
from jax.experimental import pallas as pl
from jax.experimental.pallas import tpu as pltpu
from jax.experimental.pallas import tpu_sc as plsc

N = 10000
E = 320000
D = 128
H = 4
OC = 32
NET = 8
NNT = 8
NEG = 0.2

NP = 10240
NC = 2
NS = 16
RPT = NP // NS
NW = NC * NS
EPW = E // NW
CH = 80
NCHUNK = EPW // CH

F32 = jnp.float32
I32 = jnp.int32
HIGH = jax.lax.Precision.HIGHEST


def _dotT(a, b):
    return jax.lax.dot_general(a, b, (((1,), (1,)), ((), ())), precision=HIGH)


def _dot(a, b):
    return jax.lax.dot_general(a, b, (((1,), (0,)), ((), ())), precision=HIGH)


def _pre_body(x_ref, nt_ref, wq_ref, wk_ref, wv_ref, ati_ref, atj_ref,
              ntab_ref, etab_ref, aiaj_ref, xv_ref, ea_ref, ev_ref):
    x = x_ref[...]
    nt = nt_ref[...]
    onehot = (nt == lax.broadcasted_iota(I32, (N, NNT), 1)).astype(F32)
    xp = x + _dot(onehot, ntab_ref[...])

    row = lax.broadcasted_iota(I32, (H * OC, H), 0)
    col = lax.broadcasted_iota(I32, (H * OC, H), 1)
    blkmask = ((row // OC) == col).astype(F32)
    AiM = blkmask * ati_ref[...]
    AjM = blkmask * atj_ref[...]

    Qi = jax.lax.dot_general(wq_ref[...], AiM, (((0,), (0,)), ((), ())),
                             precision=HIGH)
    Kj = jax.lax.dot_general(wk_ref[...], AjM, (((0,), (0,)), ((), ())),
                             precision=HIGH)

    ai = _dot(xp, Qi)
    aj = _dot(xp, Kj)
    aiaj_ref[...] = jnp.concatenate(
        [ai, aj, jnp.zeros((N, 8), F32)], axis=1)
    xv_ref[...] = _dotT(xp, wv_ref[...])
    ea_ref[...] = _dot(etab_ref[...], Kj)
    ev_ref[...] = _dotT(etab_ref[...], wv_ref[...])


def _precompute(x, node_type, Wq, Wk, Wv, att_i, att_j, node_table, edge_table):
    return pl.pallas_call(
        _pre_body,
        out_shape=[
            jax.ShapeDtypeStruct((N, 16), F32),
            jax.ShapeDtypeStruct((N, D), F32),
            jax.ShapeDtypeStruct((NET, H), F32),
            jax.ShapeDtypeStruct((NET, D), F32),
        ],
    )(x, node_type.reshape(N, 1), Wq, Wk, Wv,
      att_i.reshape(H * OC, 1), att_j.reshape(H * OC, 1),
      node_table, edge_table)


def _sc_body(src_hbm, dst_hbm, et_hbm, aiaj_hbm, ea_hbm, xv_hbm,
             outp_hbm, cp_hbm,
             ea_v, src_v, dst_v, et_v, aa_d, aa_s, w_v, wrow_v, rows_v,
             zv_v, zc_v, out_acc, c_acc, sem, sem_rows):
    cid = lax.axis_index("c")
    sid = lax.axis_index("s")
    wid = sid * NC + cid
    wbase = wid * EPW

    pltpu.sync_copy(ea_hbm, ea_v)

    for r in range(32):
        for j in range(8):
            zv_v[r, pl.ds(j * 16, 16)] = jnp.zeros((16,), F32)
    for r in range(32):
        for j in range(2):
            zc_v[r, pl.ds(j * 16, 16)] = jnp.zeros((16,), F32)
    for r in range(CH):
        for j in range(2):
            wrow_v[r, pl.ds(j * 16, 16)] = jnp.zeros((16,), F32)
    for i in range(RPT // 32):
        pltpu.sync_copy(zv_v, out_acc.at[pl.ds(sid * RPT + i * 32, 32)])
        pltpu.sync_copy(zc_v, c_acc.at[pl.ds(sid * RPT + i * 32, 32)])
    plsc.subcore_barrier()

    lane = lax.iota(I32, 16)

    def chunk(t, carry):
        off = wbase + t * CH
        pltpu.sync_copy(src_hbm.at[pl.ds(off, CH)], src_v)
        pltpu.sync_copy(dst_hbm.at[pl.ds(off, CH)], dst_v)
        pltpu.sync_copy(et_hbm.at[pl.ds(off, CH)], et_v)
        d_rows = pltpu.async_copy(xv_hbm.at[src_v], rows_v, sem_rows)
        d_ai = pltpu.async_copy(aiaj_hbm.at[dst_v], aa_d, sem)
        d_aj = pltpu.async_copy(aiaj_hbm.at[src_v], aa_s, sem)
        d_ai.wait()
        d_aj.wait()

        for j in range(CH // 16):
            sl = pl.ds(j * 16, 16)
            s16 = src_v[sl]
            d16 = dst_v[sl]
            e16 = et_v[sl]
            kvec = lane + j * 16
            live = s16 != d16
            for h in range(H):
                av = plsc.load_gather(aa_d, [kvec, jnp.full((16,), h, I32)])
                bv = plsc.load_gather(aa_s, [kvec, jnp.full((16,), h + H, I32)])
                gv = plsc.load_gather(ea_v, [e16 * H + h])
                a = av + bv + gv
                a = jnp.where(a >= 0, a, NEG * a)
                w = jnp.where(live, jnp.exp(a), jnp.zeros((16,), F32))
                hsplat = jnp.full((16,), h, I32)
                plsc.store_scatter(w_v, [kvec, hsplat], w)
                plsc.store_scatter(wrow_v, [kvec, e16 * H + h], w)

        d_rows.wait()

        def scale(k, carry2):
            ksplat = jnp.full((16,), k, I32)
            for h in range(H):
                wv = plsc.load_gather(w_v, [ksplat, jnp.full((16,), h, I32)])
                for j in (2 * h, 2 * h + 1):
                    cols = lane + j * 16
                    rv = plsc.load_gather(rows_v, [ksplat, cols])
                    plsc.store_scatter(rows_v, [ksplat, cols], rv * wv)
            return carry2

        lax.fori_loop(0, CH, scale, 0)

        pltpu.sync_copy(rows_v, out_acc.at[dst_v], add=True)
        pltpu.sync_copy(wrow_v, c_acc.at[dst_v], add=True)

        for j in range(CH // 16):
            sl = pl.ds(j * 16, 16)
            e16 = et_v[sl]
            kvec = lane + j * 16
            z16 = jnp.zeros((16,), F32)
            for h in range(H):
                plsc.store_scatter(wrow_v, [kvec, e16 * H + h], z16)
        return carry

    lax.fori_loop(0, NCHUNK, chunk, 0)
    plsc.subcore_barrier()

    pltpu.sync_copy(out_acc.at[pl.ds(sid * RPT, RPT)],
                    outp_hbm.at[cid, pl.ds(sid * RPT, RPT)])
    pltpu.sync_copy(c_acc.at[pl.ds(sid * RPT, RPT)],
                    cp_hbm.at[cid, pl.ds(sid * RPT, RPT)])


def _edge_phase(src, dst, et, aiaj, ea_flat, xv):
    mesh = plsc.VectorSubcoreMesh(core_axis_name="c", subcore_axis_name="s",
                                  num_cores=NC, num_subcores=NS)
    fn = pl.kernel(
        _sc_body,
        out_type=[
            jax.ShapeDtypeStruct((NC, NP, D), F32),
            jax.ShapeDtypeStruct((NC, NP, NET * H), F32),
        ],
        mesh=mesh,
        compiler_params=pltpu.CompilerParams(use_tc_tiling_on_sc=False,
                                             needs_layout_passes=False),
        scratch_types=[
            pltpu.VMEM((NET * H,), F32),
            pltpu.VMEM((CH,), I32),
            pltpu.VMEM((CH,), I32),
            pltpu.VMEM((CH,), I32),
            pltpu.VMEM((CH, 16), F32),
            pltpu.VMEM((CH, 16), F32),
            pltpu.VMEM((CH, H), F32),
            pltpu.VMEM((CH, NET * H), F32),
            pltpu.VMEM((CH, D), F32),
            pltpu.VMEM((32, D), F32),
            pltpu.VMEM((32, NET * H), F32),
            pltpu.VMEM_SHARED((NP, D), F32),
            pltpu.VMEM_SHARED((NP, NET * H), F32),
            pltpu.SemaphoreType.DMA,
            pltpu.SemaphoreType.DMA,
        ],
    )
    return fn(src, dst, et, aiaj, ea_flat, xv)


MB = 2000


def _merge_body(p_ref, c_ref, aiaj_ref, xv_ref, ea_ref, ev_ref,
                bias_ref, out_ref):
    csum = c_ref[0] + c_ref[1]
    ii = lax.broadcasted_iota(I32, (NET * H, D), 0)
    jj = lax.broadcasted_iota(I32, (NET * H, D), 1)
    HM = ((ii % H) == (jj // OC)).astype(F32)
    ri = lax.broadcasted_iota(I32, (NET * H, NET), 0)
    rj = lax.broadcasted_iota(I32, (NET * H, NET), 1)
    REP = ((ri // H) == rj).astype(F32)
    rep_ev = _dot(REP, ev_ref[...])
    M = HM * rep_ev

    out2 = _dot(csum, M)
    sfull = _dot(csum, HM)

    e4i = lax.broadcasted_iota(I32, (H, D), 0)
    e4j = lax.broadcasted_iota(I32, (H, D), 1)
    E4 = (e4i == (e4j // OC)).astype(F32)
    aiF = _dot(aiaj_ref[:, 0:H], E4)
    ajF = _dot(aiaj_ref[:, H:2 * H], E4)
    ea0F = _dot(ea_ref[0:1, :], E4)

    aself = aiF + ajF + ea0F
    aself = jnp.where(aself >= 0, aself, NEG * aself)
    wself = jnp.exp(aself)

    xv = xv_ref[...]
    ev0 = ev_ref[0:1, :]
    numer = p_ref[0] + p_ref[1] + out2 + wself * (xv + ev0)
    denom = sfull + wself + 1e-16
    out_ref[...] = numer / denom + bias_ref[...]


def _merge(p, c, aiaj, xv, ea, ev, bias2d):
    return pl.pallas_call(
        _merge_body,
        grid=(N // MB,),
        in_specs=[
            pl.BlockSpec((NC, MB, D), lambda i: (0, i, 0)),
            pl.BlockSpec((NC, MB, NET * H), lambda i: (0, i, 0)),
            pl.BlockSpec((MB, 16), lambda i: (i, 0)),
            pl.BlockSpec((MB, D), lambda i: (i, 0)),
            pl.BlockSpec((NET, H), lambda i: (0, 0)),
            pl.BlockSpec((NET, D), lambda i: (0, 0)),
            pl.BlockSpec((1, D), lambda i: (0, 0)),
        ],
        out_specs=pl.BlockSpec((MB, D), lambda i: (i, 0)),
        out_shape=jax.ShapeDtypeStruct((N, D), F32),
    )(p[:, :N, :], c[:, :N, :], aiaj, xv, ea, ev, bias2d)


def kernel(x, edge_index, node_type, edge_type, Wq, Wk, Wv, att_i, att_j,
           bias, node_table, edge_table):
    aiaj, xv, ea, ev = _precompute(
        x, node_type, Wq, Wk, Wv, att_i, att_j, node_table, edge_table)
    p, c = _edge_phase(
        edge_index[0], edge_index[1], edge_type,
        aiaj, ea.reshape(-1), xv)
    return _merge(p, c, aiaj, xv, ea, ev, bias.reshape(1, D))


# speedup vs baseline: 2.7132x; 1.2435x over previous
import functools

import jax
import jax.numpy as jnp
from jax import lax
from jax.experimental import pallas as pl
from jax.experimental.pallas import tpu as pltpu
from jax.experimental.pallas import tpu_sc as plsc

N = 10000
E = 320000
D = 128
H = 4
OC = 32
NET = 8
NNT = 8
NEG = 0.2

NP = 10240
NC = 2
NS = 16
RPT = NP // NS
NW = NC * NS
EPW = E // NW
CH = 80
NCHUNK = EPW // CH

F32 = jnp.float32
I32 = jnp.int32
HIGH = jax.lax.Precision.HIGHEST


def _dotT(a, b):
    return jax.lax.dot_general(a, b, (((1,), (1,)), ((), ())), precision=HIGH)


def _dot(a, b):
    return jax.lax.dot_general(a, b, (((1,), (0,)), ((), ())), precision=HIGH)


def _pre_body(x_ref, nt_ref, wq_ref, wk_ref, wv_ref, ati_ref, atj_ref,
              ntab_ref, etab_ref, aiaj_ref, xv_ref, ea_ref, ev_ref):
    x = x_ref[...]
    nt = nt_ref[...]
    onehot = (nt == lax.broadcasted_iota(I32, (N, NNT), 1)).astype(F32)
    xp = x + _dot(onehot, ntab_ref[...])

    row = lax.broadcasted_iota(I32, (H * OC, H), 0)
    col = lax.broadcasted_iota(I32, (H * OC, H), 1)
    blkmask = ((row // OC) == col).astype(F32)
    AiM = blkmask * ati_ref[...]
    AjM = blkmask * atj_ref[...]

    Qi = jax.lax.dot_general(wq_ref[...], AiM, (((0,), (0,)), ((), ())),
                             precision=HIGH)
    Kj = jax.lax.dot_general(wk_ref[...], AjM, (((0,), (0,)), ((), ())),
                             precision=HIGH)

    ai = _dot(xp, Qi)
    aj = _dot(xp, Kj)
    aiaj_ref[...] = jnp.concatenate(
        [ai, aj, jnp.zeros((N, 8), F32)], axis=1)
    xv_ref[...] = _dotT(xp, wv_ref[...])
    ea_ref[...] = _dot(etab_ref[...], Kj)
    ev_ref[...] = _dotT(etab_ref[...], wv_ref[...])


def _precompute(x, node_type, Wq, Wk, Wv, att_i, att_j, node_table, edge_table):
    return pl.pallas_call(
        _pre_body,
        out_shape=[
            jax.ShapeDtypeStruct((N, 16), F32),
            jax.ShapeDtypeStruct((N, D), F32),
            jax.ShapeDtypeStruct((NET, H), F32),
            jax.ShapeDtypeStruct((NET, D), F32),
        ],
    )(x, node_type.reshape(N, 1), Wq, Wk, Wv,
      att_i.reshape(H * OC, 1), att_j.reshape(H * OC, 1),
      node_table, edge_table)


def _sc_body(src_hbm, dst_hbm, et_hbm, aiaj_hbm, ea_hbm, xv_hbm,
             outp_hbm, cp_hbm,
             ea_v, srcA, dstA, etA, srcB, dstB, etB, aa_d, aa_s, w_v,
             wrow_v, rows_v, zv_v, zc_v, out_acc, c_acc,
             sem, sem_idx, sem_rows):
    cid = lax.axis_index("c")
    sid = lax.axis_index("s")
    wid = sid * NC + cid
    wbase = wid * EPW

    pltpu.sync_copy(ea_hbm, ea_v)

    for r in range(32):
        for j in range(8):
            zv_v[r, pl.ds(j * 16, 16)] = jnp.zeros((16,), F32)
    for r in range(32):
        for j in range(2):
            zc_v[r, pl.ds(j * 16, 16)] = jnp.zeros((16,), F32)
    for r in range(CH):
        for j in range(2):
            wrow_v[r, pl.ds(j * 16, 16)] = jnp.zeros((16,), F32)
    for i in range(RPT // 32):
        pltpu.sync_copy(zv_v, out_acc.at[pl.ds(sid * RPT + i * 32, 32)])
        pltpu.sync_copy(zc_v, c_acc.at[pl.ds(sid * RPT + i * 32, 32)])
    plsc.subcore_barrier()

    lane = lax.iota(I32, 16)

    def issue_idx(t, s_v, d_v, e_v):
        off = wbase + t * CH
        pltpu.async_copy(src_hbm.at[pl.ds(off, CH)], s_v, sem_idx)
        pltpu.async_copy(dst_hbm.at[pl.ds(off, CH)], d_v, sem_idx)
        pltpu.async_copy(et_hbm.at[pl.ds(off, CH)], e_v, sem_idx)

    def wait_idx(s_v, d_v, e_v):
        pltpu.make_async_copy(src_hbm.at[pl.ds(0, CH)], s_v, sem_idx).wait()
        pltpu.make_async_copy(dst_hbm.at[pl.ds(0, CH)], d_v, sem_idx).wait()
        pltpu.make_async_copy(et_hbm.at[pl.ds(0, CH)], e_v, sem_idx).wait()

    def issue_gathers(s_v, d_v):
        pltpu.async_copy(aiaj_hbm.at[d_v], aa_d, sem)
        pltpu.async_copy(aiaj_hbm.at[s_v], aa_s, sem)

    def chunk_body(t, srcC, dstC, etC, srcN, dstN, etN, issue_next):
        pltpu.make_async_copy(aiaj_hbm.at[dstC], aa_d, sem).wait()
        pltpu.make_async_copy(aiaj_hbm.at[srcC], aa_s, sem).wait()

        for j in range(CH // 16):
            sl = pl.ds(j * 16, 16)
            s16 = srcC[sl]
            d16 = dstC[sl]
            e16 = etC[sl]
            kvec = lane + j * 16
            live = s16 != d16
            for h in range(H):
                av = plsc.load_gather(aa_d, [kvec, jnp.full((16,), h, I32)])
                bv = plsc.load_gather(aa_s, [kvec, jnp.full((16,), h + H, I32)])
                gv = plsc.load_gather(ea_v, [e16 * H + h])
                a = av + bv + gv
                a = jnp.where(a >= 0, a, NEG * a)
                w = jnp.where(live, jnp.exp(a), jnp.zeros((16,), F32))
                hsplat = jnp.full((16,), h, I32)
                plsc.store_scatter(w_v, [kvec, hsplat], w)
                plsc.store_scatter(wrow_v, [kvec, e16 * H + h], w)

        if issue_next:
            issue_idx(t + 1, srcN, dstN, etN)

        pltpu.make_async_copy(xv_hbm.at[srcC], rows_v, sem_rows).wait()

        def scale(k, carry2):
            ksplat = jnp.full((16,), k, I32)
            for h in range(H):
                wv = plsc.load_gather(w_v, [ksplat, jnp.full((16,), h, I32)])
                for j in (2 * h, 2 * h + 1):
                    cols = lane + j * 16
                    rv = plsc.load_gather(rows_v, [ksplat, cols])
                    plsc.store_scatter(rows_v, [ksplat, cols], rv * wv)
            return carry2

        lax.fori_loop(0, CH, scale, 0)

        if issue_next:
            wait_idx(srcN, dstN, etN)
            issue_gathers(srcN, dstN)

        pltpu.sync_copy(rows_v, out_acc.at[dstC], add=True)
        pltpu.sync_copy(wrow_v, c_acc.at[dstC], add=True)

        for j in range(CH // 16):
            sl = pl.ds(j * 16, 16)
            e16 = etC[sl]
            kvec = lane + j * 16
            z16 = jnp.zeros((16,), F32)
            for h in range(H):
                plsc.store_scatter(wrow_v, [kvec, e16 * H + h], z16)

        if issue_next:
            pltpu.async_copy(xv_hbm.at[srcN], rows_v, sem_rows)

    issue_idx(0, srcA, dstA, etA)
    wait_idx(srcA, dstA, etA)
    issue_gathers(srcA, dstA)
    pltpu.async_copy(xv_hbm.at[srcA], rows_v, sem_rows)

    def pair(i, carry):
        t0 = 2 * i
        chunk_body(t0, srcA, dstA, etA, srcB, dstB, etB, True)
        chunk_body(t0 + 1, srcB, dstB, etB, srcA, dstA, etA, True)
        return carry

    lax.fori_loop(0, (NCHUNK - 1) // 2, pair, 0)
    chunk_body(NCHUNK - 1, srcA, dstA, etA, srcB, dstB, etB, False)
    plsc.subcore_barrier()

    pltpu.sync_copy(out_acc.at[pl.ds(sid * RPT, RPT)],
                    outp_hbm.at[cid, pl.ds(sid * RPT, RPT)])
    pltpu.sync_copy(c_acc.at[pl.ds(sid * RPT, RPT)],
                    cp_hbm.at[cid, pl.ds(sid * RPT, RPT)])


def _edge_phase(src, dst, et, aiaj, ea_flat, xv):
    mesh = plsc.VectorSubcoreMesh(core_axis_name="c", subcore_axis_name="s",
                                  num_cores=NC, num_subcores=NS)
    fn = pl.kernel(
        _sc_body,
        out_type=[
            jax.ShapeDtypeStruct((NC, NP, D), F32),
            jax.ShapeDtypeStruct((NC, NP, NET * H), F32),
        ],
        mesh=mesh,
        compiler_params=pltpu.CompilerParams(use_tc_tiling_on_sc=False,
                                             needs_layout_passes=False),
        scratch_types=[
            pltpu.VMEM((NET * H,), F32),
            pltpu.VMEM((CH,), I32),
            pltpu.VMEM((CH,), I32),
            pltpu.VMEM((CH,), I32),
            pltpu.VMEM((CH,), I32),
            pltpu.VMEM((CH,), I32),
            pltpu.VMEM((CH,), I32),
            pltpu.VMEM((CH, 16), F32),
            pltpu.VMEM((CH, 16), F32),
            pltpu.VMEM((CH, H), F32),
            pltpu.VMEM((CH, NET * H), F32),
            pltpu.VMEM((CH, D), F32),
            pltpu.VMEM((32, D), F32),
            pltpu.VMEM((32, NET * H), F32),
            pltpu.VMEM_SHARED((NP, D), F32),
            pltpu.VMEM_SHARED((NP, NET * H), F32),
            pltpu.SemaphoreType.DMA,
            pltpu.SemaphoreType.DMA,
            pltpu.SemaphoreType.DMA,
        ],
    )
    return fn(src, dst, et, aiaj, ea_flat, xv)


MB = 2000


def _merge_body(p_ref, c_ref, aiaj_ref, xv_ref, ea_ref, ev_ref,
                bias_ref, out_ref):
    csum = c_ref[0] + c_ref[1]
    ii = lax.broadcasted_iota(I32, (NET * H, D), 0)
    jj = lax.broadcasted_iota(I32, (NET * H, D), 1)
    HM = ((ii % H) == (jj // OC)).astype(F32)
    ri = lax.broadcasted_iota(I32, (NET * H, NET), 0)
    rj = lax.broadcasted_iota(I32, (NET * H, NET), 1)
    REP = ((ri // H) == rj).astype(F32)
    rep_ev = _dot(REP, ev_ref[...])
    M = HM * rep_ev

    out2 = _dot(csum, M)
    sfull = _dot(csum, HM)

    e4i = lax.broadcasted_iota(I32, (H, D), 0)
    e4j = lax.broadcasted_iota(I32, (H, D), 1)
    E4 = (e4i == (e4j // OC)).astype(F32)
    aiF = _dot(aiaj_ref[:, 0:H], E4)
    ajF = _dot(aiaj_ref[:, H:2 * H], E4)
    ea0F = _dot(ea_ref[0:1, :], E4)

    aself = aiF + ajF + ea0F
    aself = jnp.where(aself >= 0, aself, NEG * aself)
    wself = jnp.exp(aself)

    xv = xv_ref[...]
    ev0 = ev_ref[0:1, :]
    numer = p_ref[0] + p_ref[1] + out2 + wself * (xv + ev0)
    denom = sfull + wself + 1e-16
    out_ref[...] = numer / denom + bias_ref[...]


def _merge(p, c, aiaj, xv, ea, ev, bias2d):
    return pl.pallas_call(
        _merge_body,
        grid=(N // MB,),
        in_specs=[
            pl.BlockSpec((NC, MB, D), lambda i: (0, i, 0)),
            pl.BlockSpec((NC, MB, NET * H), lambda i: (0, i, 0)),
            pl.BlockSpec((MB, 16), lambda i: (i, 0)),
            pl.BlockSpec((MB, D), lambda i: (i, 0)),
            pl.BlockSpec((NET, H), lambda i: (0, 0)),
            pl.BlockSpec((NET, D), lambda i: (0, 0)),
            pl.BlockSpec((1, D), lambda i: (0, 0)),
        ],
        out_specs=pl.BlockSpec((MB, D), lambda i: (i, 0)),
        out_shape=jax.ShapeDtypeStruct((N, D), F32),
    )(p[:, :N, :], c[:, :N, :], aiaj, xv, ea, ev, bias2d)


def kernel(x, edge_index, node_type, edge_type, Wq, Wk, Wv, att_i, att_j,
           bias, node_table, edge_table):
    aiaj, xv, ea, ev = _precompute(
        x, node_type, Wq, Wk, Wv, att_i, att_j, node_table, edge_table)
    p, c = _edge_phase(
        edge_index[0], edge_index[1], edge_type,
        aiaj, ea.reshape(-1), xv)
    return _merge(p, c, aiaj, xv, ea, ev, bias.reshape(1, D))


# speedup vs baseline: 4.9626x; 1.8290x over previous
import functools

import jax
import jax.numpy as jnp
from jax import lax
from jax.experimental import pallas as pl
from jax.experimental.pallas import tpu as pltpu
from jax.experimental.pallas import tpu_sc as plsc

N = 10000
E = 320000
D = 128
H = 4
OC = 32
NET = 8
NNT = 8
NEG = 0.2

NP = 10240
NC = 2
NS = 16
RPT = NP // NS
NW = NC * NS
EPW = E // NW
CH = 80
NCHUNK = EPW // CH

F32 = jnp.float32
I32 = jnp.int32
HIGH = jax.lax.Precision.HIGHEST


def _dotT(a, b):
    return jax.lax.dot_general(a, b, (((1,), (1,)), ((), ())), precision=HIGH)


def _dot(a, b):
    return jax.lax.dot_general(a, b, (((1,), (0,)), ((), ())), precision=HIGH)


def _pre_body(x_ref, nt_ref, wq_ref, wk_ref, wv_ref, ati_ref, atj_ref,
              ntab_ref, etab_ref, aiaj_ref, xv_ref, ea_ref, ev_ref):
    x = x_ref[...]
    nt = nt_ref[...]
    onehot = (nt == lax.broadcasted_iota(I32, (N, NNT), 1)).astype(F32)
    xp = x + _dot(onehot, ntab_ref[...])

    row = lax.broadcasted_iota(I32, (H * OC, H), 0)
    col = lax.broadcasted_iota(I32, (H * OC, H), 1)
    blkmask = ((row // OC) == col).astype(F32)
    AiM = blkmask * ati_ref[...]
    AjM = blkmask * atj_ref[...]

    Qi = jax.lax.dot_general(wq_ref[...], AiM, (((0,), (0,)), ((), ())),
                             precision=HIGH)
    Kj = jax.lax.dot_general(wk_ref[...], AjM, (((0,), (0,)), ((), ())),
                             precision=HIGH)

    ai = _dot(xp, Qi)
    aj = _dot(xp, Kj)
    aiaj_ref[...] = jnp.concatenate(
        [ai, aj, jnp.zeros((N, 8), F32)], axis=1)
    xv_ref[...] = _dotT(xp, wv_ref[...])
    ea_ref[...] = _dot(etab_ref[...], Kj)
    ev_ref[...] = _dotT(etab_ref[...], wv_ref[...])


def _precompute(x, node_type, Wq, Wk, Wv, att_i, att_j, node_table, edge_table):
    return pl.pallas_call(
        _pre_body,
        out_shape=[
            jax.ShapeDtypeStruct((N, 16), F32),
            jax.ShapeDtypeStruct((N, D), F32),
            jax.ShapeDtypeStruct((NET, H), F32),
            jax.ShapeDtypeStruct((NET, D), F32),
        ],
    )(x, node_type.reshape(N, 1), Wq, Wk, Wv,
      att_i.reshape(H * OC, 1), att_j.reshape(H * OC, 1),
      node_table, edge_table)


def _sc_body(src_hbm, dst_hbm, et_hbm, aiaj_hbm, ea_hbm, xv_hbm,
             outp_hbm, cp_hbm,
             ea_v, srcA, dstA, etA, srcB, dstB, etB, aa_d, aa_s, w_v,
             wrow_v, rows_v, zv_v, zc_v, out_acc, c_acc,
             sem, sem_idx, sem_rows):
    cid = lax.axis_index("c")
    sid = lax.axis_index("s")
    wid = sid * NC + cid
    wbase = wid * EPW

    pltpu.sync_copy(ea_hbm, ea_v)

    for r in range(32):
        for j in range(8):
            zv_v[r, pl.ds(j * 16, 16)] = jnp.zeros((16,), F32)
    for r in range(32):
        for j in range(2):
            zc_v[r, pl.ds(j * 16, 16)] = jnp.zeros((16,), F32)
    for r in range(CH):
        for j in range(2):
            wrow_v[r, pl.ds(j * 16, 16)] = jnp.zeros((16,), F32)
    for i in range(RPT // 32):
        pltpu.sync_copy(zv_v, out_acc.at[pl.ds(sid * RPT + i * 32, 32)])
        pltpu.sync_copy(zc_v, c_acc.at[pl.ds(sid * RPT + i * 32, 32)])
    plsc.subcore_barrier()

    lane = lax.iota(I32, 16)

    def issue_idx(t, s_v, d_v, e_v):
        off = wbase + t * CH
        pltpu.async_copy(src_hbm.at[pl.ds(off, CH)], s_v, sem_idx)
        pltpu.async_copy(dst_hbm.at[pl.ds(off, CH)], d_v, sem_idx)
        pltpu.async_copy(et_hbm.at[pl.ds(off, CH)], e_v, sem_idx)

    def wait_idx(s_v, d_v, e_v):
        pltpu.make_async_copy(src_hbm.at[pl.ds(0, CH)], s_v, sem_idx).wait()
        pltpu.make_async_copy(dst_hbm.at[pl.ds(0, CH)], d_v, sem_idx).wait()
        pltpu.make_async_copy(et_hbm.at[pl.ds(0, CH)], e_v, sem_idx).wait()

    def issue_gathers(s_v, d_v):
        pltpu.async_copy(aiaj_hbm.at[d_v], aa_d, sem)
        pltpu.async_copy(aiaj_hbm.at[s_v], aa_s, sem)

    def chunk_body(t, srcC, dstC, etC, srcN, dstN, etN, issue_next):
        pltpu.make_async_copy(aiaj_hbm.at[dstC], aa_d, sem).wait()
        pltpu.make_async_copy(aiaj_hbm.at[srcC], aa_s, sem).wait()

        for j in range(CH // 16):
            sl = pl.ds(j * 16, 16)
            s16 = srcC[sl]
            d16 = dstC[sl]
            e16 = etC[sl]
            kvec = lane + j * 16
            live = s16 != d16
            for h in range(H):
                av = plsc.load_gather(aa_d, [kvec, jnp.full((16,), h, I32)])
                bv = plsc.load_gather(aa_s, [kvec, jnp.full((16,), h + H, I32)])
                gv = plsc.load_gather(ea_v, [e16 * H + h])
                a = av + bv + gv
                a = jnp.where(a >= 0, a, NEG * a)
                w = jnp.where(live, jnp.exp(a), jnp.zeros((16,), F32))
                hsplat = jnp.full((16,), h, I32)
                plsc.store_scatter(w_v, [kvec, hsplat], w)
                plsc.store_scatter(wrow_v, [kvec, e16 * H + h], w)

        if issue_next:
            issue_idx(t + 1, srcN, dstN, etN)

        pltpu.make_async_copy(xv_hbm.at[srcC], rows_v, sem_rows).wait()

        @plsc.parallel_loop(0, CH, 1, unroll=4)
        def scale(k):
            ksplat = jnp.full((16,), k, I32)
            for h in range(H):
                wv = plsc.load_gather(w_v, [ksplat, jnp.full((16,), h, I32)])
                for j in (2 * h, 2 * h + 1):
                    cols = lane + j * 16
                    rv = plsc.load_gather(rows_v, [ksplat, cols])
                    plsc.store_scatter(rows_v, [ksplat, cols], rv * wv)

        if issue_next:
            wait_idx(srcN, dstN, etN)
            issue_gathers(srcN, dstN)

        pltpu.sync_copy(rows_v, out_acc.at[dstC], add=True)
        pltpu.sync_copy(wrow_v, c_acc.at[dstC], add=True)

        for j in range(CH // 16):
            sl = pl.ds(j * 16, 16)
            e16 = etC[sl]
            kvec = lane + j * 16
            z16 = jnp.zeros((16,), F32)
            for h in range(H):
                plsc.store_scatter(wrow_v, [kvec, e16 * H + h], z16)

        if issue_next:
            pltpu.async_copy(xv_hbm.at[srcN], rows_v, sem_rows)

    issue_idx(0, srcA, dstA, etA)
    wait_idx(srcA, dstA, etA)
    issue_gathers(srcA, dstA)
    pltpu.async_copy(xv_hbm.at[srcA], rows_v, sem_rows)

    def pair(i, carry):
        t0 = 2 * i
        chunk_body(t0, srcA, dstA, etA, srcB, dstB, etB, True)
        chunk_body(t0 + 1, srcB, dstB, etB, srcA, dstA, etA, True)
        return carry

    lax.fori_loop(0, (NCHUNK - 1) // 2, pair, 0)
    chunk_body(NCHUNK - 1, srcA, dstA, etA, srcB, dstB, etB, False)
    plsc.subcore_barrier()

    pltpu.sync_copy(out_acc.at[pl.ds(sid * RPT, RPT)],
                    outp_hbm.at[cid, pl.ds(sid * RPT, RPT)])
    pltpu.sync_copy(c_acc.at[pl.ds(sid * RPT, RPT)],
                    cp_hbm.at[cid, pl.ds(sid * RPT, RPT)])


def _edge_phase(src, dst, et, aiaj, ea_flat, xv):
    mesh = plsc.VectorSubcoreMesh(core_axis_name="c", subcore_axis_name="s",
                                  num_cores=NC, num_subcores=NS)
    fn = pl.kernel(
        _sc_body,
        out_type=[
            jax.ShapeDtypeStruct((NC, NP, D), F32),
            jax.ShapeDtypeStruct((NC, NP, NET * H), F32),
        ],
        mesh=mesh,
        compiler_params=pltpu.CompilerParams(use_tc_tiling_on_sc=False,
                                             needs_layout_passes=False),
        scratch_types=[
            pltpu.VMEM((NET * H,), F32),
            pltpu.VMEM((CH,), I32),
            pltpu.VMEM((CH,), I32),
            pltpu.VMEM((CH,), I32),
            pltpu.VMEM((CH,), I32),
            pltpu.VMEM((CH,), I32),
            pltpu.VMEM((CH,), I32),
            pltpu.VMEM((CH, 16), F32),
            pltpu.VMEM((CH, 16), F32),
            pltpu.VMEM((CH, H), F32),
            pltpu.VMEM((CH, NET * H), F32),
            pltpu.VMEM((CH, D), F32),
            pltpu.VMEM((32, D), F32),
            pltpu.VMEM((32, NET * H), F32),
            pltpu.VMEM_SHARED((NP, D), F32),
            pltpu.VMEM_SHARED((NP, NET * H), F32),
            pltpu.SemaphoreType.DMA,
            pltpu.SemaphoreType.DMA,
            pltpu.SemaphoreType.DMA,
        ],
    )
    return fn(src, dst, et, aiaj, ea_flat, xv)


MB = 2000


def _merge_body(p_ref, c_ref, aiaj_ref, xv_ref, ea_ref, ev_ref,
                bias_ref, out_ref):
    csum = c_ref[0] + c_ref[1]
    ii = lax.broadcasted_iota(I32, (NET * H, D), 0)
    jj = lax.broadcasted_iota(I32, (NET * H, D), 1)
    HM = ((ii % H) == (jj // OC)).astype(F32)
    ri = lax.broadcasted_iota(I32, (NET * H, NET), 0)
    rj = lax.broadcasted_iota(I32, (NET * H, NET), 1)
    REP = ((ri // H) == rj).astype(F32)
    rep_ev = _dot(REP, ev_ref[...])
    M = HM * rep_ev

    out2 = _dot(csum, M)
    sfull = _dot(csum, HM)

    e4i = lax.broadcasted_iota(I32, (H, D), 0)
    e4j = lax.broadcasted_iota(I32, (H, D), 1)
    E4 = (e4i == (e4j // OC)).astype(F32)
    aiF = _dot(aiaj_ref[:, 0:H], E4)
    ajF = _dot(aiaj_ref[:, H:2 * H], E4)
    ea0F = _dot(ea_ref[0:1, :], E4)

    aself = aiF + ajF + ea0F
    aself = jnp.where(aself >= 0, aself, NEG * aself)
    wself = jnp.exp(aself)

    xv = xv_ref[...]
    ev0 = ev_ref[0:1, :]
    numer = p_ref[0] + p_ref[1] + out2 + wself * (xv + ev0)
    denom = sfull + wself + 1e-16
    out_ref[...] = numer / denom + bias_ref[...]


def _merge(p, c, aiaj, xv, ea, ev, bias2d):
    return pl.pallas_call(
        _merge_body,
        grid=(N // MB,),
        in_specs=[
            pl.BlockSpec((NC, MB, D), lambda i: (0, i, 0)),
            pl.BlockSpec((NC, MB, NET * H), lambda i: (0, i, 0)),
            pl.BlockSpec((MB, 16), lambda i: (i, 0)),
            pl.BlockSpec((MB, D), lambda i: (i, 0)),
            pl.BlockSpec((NET, H), lambda i: (0, 0)),
            pl.BlockSpec((NET, D), lambda i: (0, 0)),
            pl.BlockSpec((1, D), lambda i: (0, 0)),
        ],
        out_specs=pl.BlockSpec((MB, D), lambda i: (i, 0)),
        out_shape=jax.ShapeDtypeStruct((N, D), F32),
    )(p[:, :N, :], c[:, :N, :], aiaj, xv, ea, ev, bias2d)


def kernel(x, edge_index, node_type, edge_type, Wq, Wk, Wv, att_i, att_j,
           bias, node_table, edge_table):
    aiaj, xv, ea, ev = _precompute(
        x, node_type, Wq, Wk, Wv, att_i, att_j, node_table, edge_table)
    p, c = _edge_phase(
        edge_index[0], edge_index[1], edge_type,
        aiaj, ea.reshape(-1), xv)
    return _merge(p, c, aiaj, xv, ea, ev, bias.reshape(1, D))
